# baseline (speedup 1.0000x reference)
import functools

import jax
import jax.numpy as jnp
from jax import lax
from jax.experimental import pallas as pl
from jax.experimental.pallas import tpu as pltpu
from jax.experimental.pallas import tpu_sc as plsc

B = 16384
EMB = 16
CONTENT_DIM = 128
H1, H2 = 64, 32

_info = plsc.get_sparse_core_info()
_NC, _NS = _info.num_cores, _info.num_subcores
_NW = _NC * _NS
_BPW = B // _NW


def _sc_gather(users, items, user_gmf, item_gmf, user_mlp, item_mlp):
    mesh = plsc.VectorSubcoreMesh(core_axis_name="c", subcore_axis_name="s")

    @functools.partial(
        pl.kernel,
        mesh=mesh,
        out_type=[jax.ShapeDtypeStruct((B, EMB), jnp.float32)] * 4,
        scratch_types=[
            pltpu.VMEM((_BPW,), jnp.int32),
            pltpu.VMEM((_BPW,), jnp.int32),
            pltpu.SemaphoreType.DMA,
        ],
    )
    def k(users_hbm, items_hbm, ug_hbm, ig_hbm, um_hbm, im_hbm,
          gu_out, gi_out, mu_out, mi_out, uidx, iidx, sem):
        wid = lax.axis_index("s") * _NC + lax.axis_index("c")
        base = wid * _BPW
        pltpu.sync_copy(users_hbm.at[pl.ds(base, _BPW)], uidx)
        pltpu.sync_copy(items_hbm.at[pl.ds(base, _BPW)], iidx)

        def issue(g, carry):
            uvec = uidx[pl.ds(g * 16, 16)]
            ivec = iidx[pl.ds(g * 16, 16)]
            row0 = base + g * 16
            for j in range(16):
                u = uvec[j]
                v = ivec[j]
                pltpu.async_copy(ug_hbm.at[u], gu_out.at[row0 + j], sem)
                pltpu.async_copy(ig_hbm.at[v], gi_out.at[row0 + j], sem)
                pltpu.async_copy(um_hbm.at[u], mu_out.at[row0 + j], sem)
                pltpu.async_copy(im_hbm.at[v], mi_out.at[row0 + j], sem)
            return carry

        lax.fori_loop(0, _BPW // 16, issue, 0)

        for out in (gu_out, gi_out, mu_out, mi_out):
            pltpu.make_async_copy(out.at[pl.ds(base, _BPW)],
                                  out.at[pl.ds(base, _BPW)], sem).wait()

    return k(users, items, user_gmf, item_gmf, user_mlp, item_mlp)


def _tc_mlp(content_vec, gu, gi, mu, mi, W_content, W1, b1, W2, b2, W_out):
    WcT = W_content.T
    W1T = W1.T
    W1u, W1i, W1p = W1T[0:EMB], W1T[EMB:2 * EMB], W1T[2 * EMB:3 * EMB]
    W2T = W2.T
    wg = W_out[:, 0:EMB]
    wh = W_out[:, EMB:]
    b1r = b1.reshape(1, H1)
    b2r = b2.reshape(1, H2)

    BLK = 2048
    grid = (B // BLK,)
    row = lambda i: (i, 0)
    rep = lambda i: (0, 0)

    def body(c_ref, gu_ref, gi_ref, mu_ref, mi_ref,
             wc_ref, w1u_ref, w1i_ref, w1p_ref, b1_ref, w2_ref, b2_ref,
             wg_ref, wh_ref, out_ref):
        proj = jnp.maximum(
            jnp.dot(c_ref[...], wc_ref[...], preferred_element_type=jnp.float32), 0.0)
        pre1 = (jnp.dot(mu_ref[...], w1u_ref[...], preferred_element_type=jnp.float32)
                + jnp.dot(mi_ref[...], w1i_ref[...], preferred_element_type=jnp.float32)
                + jnp.dot(proj, w1p_ref[...], preferred_element_type=jnp.float32)
                + b1_ref[...])
        h1 = jnp.maximum(pre1, 0.0)
        h2 = jnp.maximum(
            jnp.dot(h1, w2_ref[...], preferred_element_type=jnp.float32) + b2_ref[...],
            0.0)
        gmf = gu_ref[...] * gi_ref[...]
        logits = (jnp.sum(gmf * wg_ref[...], axis=1, keepdims=True)
                  + jnp.sum(h2 * wh_ref[...], axis=1, keepdims=True))
        out_ref[...] = jax.nn.sigmoid(logits)

    out = pl.pallas_call(
        body,
        grid=grid,
        in_specs=[
            pl.BlockSpec((BLK, CONTENT_DIM), row),
            pl.BlockSpec((BLK, EMB), row),
            pl.BlockSpec((BLK, EMB), row),
            pl.BlockSpec((BLK, EMB), row),
            pl.BlockSpec((BLK, EMB), row),
            pl.BlockSpec((CONTENT_DIM, EMB), rep),
            pl.BlockSpec((EMB, H1), rep),
            pl.BlockSpec((EMB, H1), rep),
            pl.BlockSpec((EMB, H1), rep),
            pl.BlockSpec((1, H1), rep),
            pl.BlockSpec((H1, H2), rep),
            pl.BlockSpec((1, H2), rep),
            pl.BlockSpec((1, EMB), rep),
            pl.BlockSpec((1, H2), rep),
        ],
        out_specs=pl.BlockSpec((BLK, 1), row),
        out_shape=jax.ShapeDtypeStruct((B, 1), jnp.float32),
    )(content_vec, gu, gi, mu, mi, WcT, W1u, W1i, W1p, b1r, W2T, b2r, wg, wh)
    return out[:, 0]


def kernel(users, items, content_vec, user_gmf, item_gmf, user_mlp, item_mlp,
           W_content, W1, b1, W2, b2, W_out):
    users = users.astype(jnp.int32)
    items = items.astype(jnp.int32)
    gu, gi, mu, mi = _sc_gather(users, items, user_gmf, item_gmf,
                                user_mlp, item_mlp)
    return _tc_mlp(content_vec, gu, gi, mu, mi, W_content, W1, b1, W2, b2, W_out)


# speedup vs baseline: 1.9120x; 1.9120x over previous
import functools

import jax
import jax.numpy as jnp
from jax import lax
from jax.experimental import pallas as pl
from jax.experimental.pallas import tpu as pltpu
from jax.experimental.pallas import tpu_sc as plsc

B = 16384
EMB = 16
CONTENT_DIM = 128
H1, H2 = 64, 32
N_ROWS = 1000000

_info = plsc.get_sparse_core_info()
_NC, _NS = _info.num_cores, _info.num_subcores
_NW = _NC * _NS
_BPW = B // _NW


_CH = 16
_NCHUNK = _BPW // _CH


def _sc_gather(users, items, user_gmf, item_gmf, user_mlp, item_mlp):
    mesh = plsc.VectorSubcoreMesh(core_axis_name="c", subcore_axis_name="s")

    @functools.partial(
        pl.kernel,
        mesh=mesh,
        out_type=[jax.ShapeDtypeStruct((B, EMB), jnp.float32)] * 4,
        scratch_types=[
            pltpu.VMEM((_BPW,), jnp.int32),
            pltpu.VMEM((_BPW,), jnp.int32),
            pltpu.VMEM((_BPW, EMB), jnp.float32),
            pltpu.SemaphoreType.DMA,
        ],
    )
    def k(users_hbm, items_hbm, ug_hbm, ig_hbm, um_hbm, im_hbm,
          gu_out, gi_out, mu_out, mi_out, uidx, iidx, rows_v, sem):
        wid = lax.axis_index("s") * _NC + lax.axis_index("c")
        base = wid * _BPW
        pltpu.sync_copy(users_hbm.at[pl.ds(base, _BPW)], uidx)
        pltpu.sync_copy(items_hbm.at[pl.ds(base, _BPW)], iidx)

        for table_hbm, idx_v, out_hbm in (
                (ug_hbm, uidx, gu_out),
                (ig_hbm, iidx, gi_out),
                (um_hbm, uidx, mu_out),
                (im_hbm, iidx, mi_out)):

            def issue(g, carry):
                vec = idx_v[pl.ds(g * 16, 16)]
                for j in range(16):
                    pltpu.async_copy(table_hbm.at[vec[j]],
                                     rows_v.at[g * 16 + j], sem)
                return carry

            lax.fori_loop(0, _BPW // 16, issue, 0)
            pltpu.make_async_copy(table_hbm.at[pl.ds(0, _BPW)],
                                  rows_v, sem).wait()
            pltpu.sync_copy(rows_v, out_hbm.at[pl.ds(base, _BPW)])

    return k(users, items, user_gmf, item_gmf, user_mlp, item_mlp)


def _tc_mlp(content_vec, gu, gi, mu, mi, W_content, W1, b1, W2, b2, W_out):
    WcT = W_content.T
    W1T = W1.T
    W1u, W1i, W1p = W1T[0:EMB], W1T[EMB:2 * EMB], W1T[2 * EMB:3 * EMB]
    W2T = W2.T
    wg = W_out[:, 0:EMB]
    wh = W_out[:, EMB:]
    b1r = b1.reshape(1, H1)
    b2r = b2.reshape(1, H2)

    BLK = 2048
    grid = (B // BLK,)
    row = lambda i: (i, 0)
    rep = lambda i: (0, 0)

    def body(c_ref, gu_ref, gi_ref, mu_ref, mi_ref,
             wc_ref, w1u_ref, w1i_ref, w1p_ref, b1_ref, w2_ref, b2_ref,
             wg_ref, wh_ref, out_ref):
        proj = jnp.maximum(
            jnp.dot(c_ref[...], wc_ref[...], preferred_element_type=jnp.float32), 0.0)
        pre1 = (jnp.dot(mu_ref[...], w1u_ref[...], preferred_element_type=jnp.float32)
                + jnp.dot(mi_ref[...], w1i_ref[...], preferred_element_type=jnp.float32)
                + jnp.dot(proj, w1p_ref[...], preferred_element_type=jnp.float32)
                + b1_ref[...])
        h1 = jnp.maximum(pre1, 0.0)
        h2 = jnp.maximum(
            jnp.dot(h1, w2_ref[...], preferred_element_type=jnp.float32) + b2_ref[...],
            0.0)
        gmf = gu_ref[...] * gi_ref[...]
        logits = (jnp.sum(gmf * wg_ref[...], axis=1, keepdims=True)
                  + jnp.sum(h2 * wh_ref[...], axis=1, keepdims=True))
        out_ref[...] = jax.nn.sigmoid(logits)

    out = pl.pallas_call(
        body,
        grid=grid,
        in_specs=[
            pl.BlockSpec((BLK, CONTENT_DIM), row),
            pl.BlockSpec((BLK, EMB), row),
            pl.BlockSpec((BLK, EMB), row),
            pl.BlockSpec((BLK, EMB), row),
            pl.BlockSpec((BLK, EMB), row),
            pl.BlockSpec((CONTENT_DIM, EMB), rep),
            pl.BlockSpec((EMB, H1), rep),
            pl.BlockSpec((EMB, H1), rep),
            pl.BlockSpec((EMB, H1), rep),
            pl.BlockSpec((1, H1), rep),
            pl.BlockSpec((H1, H2), rep),
            pl.BlockSpec((1, H2), rep),
            pl.BlockSpec((1, EMB), rep),
            pl.BlockSpec((1, H2), rep),
        ],
        out_specs=pl.BlockSpec((BLK, 1), row),
        out_shape=jax.ShapeDtypeStruct((B, 1), jnp.float32),
    )(content_vec, gu, gi, mu, mi, WcT, W1u, W1i, W1p, b1r, W2T, b2r, wg, wh)
    return out[:, 0]


def kernel(users, items, content_vec, user_gmf, item_gmf, user_mlp, item_mlp,
           W_content, W1, b1, W2, b2, W_out):
    users = users.astype(jnp.int32)
    items = items.astype(jnp.int32)
    gu, gi, mu, mi = _sc_gather(users, items, user_gmf, item_gmf,
                                user_mlp, item_mlp)
    return _tc_mlp(content_vec, gu, gi, mu, mi, W_content, W1, b1, W2, b2, W_out)
